# baseline (speedup 1.0000x reference)
import functools

import jax
import jax.numpy as jnp
from jax import lax
from jax.experimental import pallas as pl
from jax.experimental.pallas import tpu as pltpu
from jax.experimental.pallas import tpu_sc as plsc

N = 10000
E = 320000
D = 128

NC = 2
NS = 16
NPAD = 10240
RPT = NPAD // NS
K = 80

KC = 128
RBUF = 4
SPB = 2
PADE = 327680

DG_CH = PADE // (NC * NS * KC)
SP_CH = PADE // (NS * KC)


def _deg_body(dst3_hbm, ones_hbm, z_hbm, degp_hbm, dall, ones_v, acc,
              s0, s1, s2, s3):
    cid = lax.axis_index("c")
    sid = lax.axis_index("s")
    wid = cid * NS + sid
    sems = (s0, s1, s2, s3)
    pltpu.sync_copy(ones_hbm, ones_v)
    pltpu.sync_copy(dst3_hbm.at[wid], dall)
    pltpu.sync_copy(z_hbm, acc.at[pl.ds(sid * RPT, RPT)])
    plsc.subcore_barrier()

    def outer(j, _):
        descs = []
        for b in range(RBUF):
            i = j * RBUF + b
            descs.append(pltpu.async_copy(ones_v, acc.at[dall.at[i]],
                                          sems[b], add=True))
        for d in descs:
            d.wait()
        return 0

    lax.fori_loop(0, DG_CH // RBUF, outer, 0)
    plsc.subcore_barrier()
    pltpu.sync_copy(acc.at[pl.ds(sid * RPT, RPT)],
                    degp_hbm.at[cid, pl.ds(sid * RPT, RPT)])


def _sc_degree(dst3):
    mesh = plsc.VectorSubcoreMesh(core_axis_name="c", subcore_axis_name="s")
    ones = jnp.ones((KC, D), jnp.float32)
    zeros = jnp.zeros((RPT, D), jnp.float32)
    kern = pl.kernel(
        _deg_body,
        out_type=jax.ShapeDtypeStruct((NC, NPAD, D), jnp.float32),
        mesh=mesh,
        scratch_types=[
            pltpu.VMEM((DG_CH, KC), jnp.int32),
            pltpu.VMEM((KC, D), jnp.float32),
            pltpu.VMEM_SHARED((NPAD, D), jnp.float32),
            pltpu.SemaphoreType.DMA,
            pltpu.SemaphoreType.DMA,
            pltpu.SemaphoreType.DMA,
            pltpu.SemaphoreType.DMA,
        ],
    )
    return kern(dst3, ones, zeros)


GRP = 16
SKC = 64
SP_CH2 = PADE // (NS * SKC)


def _spmm_one(tab_hbm, src2_hbm, dst2_hbm, z_hbm, out_hbm,
              sblk, dblk, rows, acc, isem, gsems, ssems, sid):
    pltpu.sync_copy(z_hbm, acc.at[pl.ds(sid * RPT, RPT)])
    plsc.subcore_barrier()
    cbase = sid * SP_CH2

    def outer(j, _):
        c0 = cbase + j * GRP
        i1 = pltpu.async_copy(src2_hbm.at[pl.ds(c0, GRP)], sblk, isem)
        i2 = pltpu.async_copy(dst2_hbm.at[pl.ds(c0, GRP)], dblk, isem)
        i1.wait()
        i2.wait()
        sds = [None, None]
        gds = [None, None]
        gds[0] = pltpu.async_copy(tab_hbm.at[sblk.at[0]], rows[0], gsems[0])
        for b in range(GRP):
            r = b % 2
            nr = (b + 1) % 2
            if b + 1 < GRP:
                if sds[nr] is not None:
                    sds[nr].wait()
                    sds[nr] = None
                gds[nr] = pltpu.async_copy(tab_hbm.at[sblk.at[b + 1]],
                                           rows[nr], gsems[nr])
            gds[r].wait()
            sds[r] = pltpu.async_copy(rows[r], acc.at[dblk.at[b]],
                                      ssems[r], add=True)
        for d in sds:
            d.wait()
        return 0

    lax.fori_loop(0, SP_CH2 // GRP, outer, 0)
    plsc.subcore_barrier()
    pltpu.sync_copy(acc.at[pl.ds(sid * RPT, RPT)],
                    out_hbm.at[pl.ds(sid * RPT, RPT)])


def _spmm_body(a_hbm, b_hbm, src2_hbm, dst2_hbm, z_hbm, s1_hbm, s2_hbm,
               sblk, dblk, r0, r1, acc,
               isem, g0, g1, t0, t1):
    cid = lax.axis_index("c")
    sid = lax.axis_index("s")
    rows = (r0, r1)
    gsems = (g0, g1)
    ssems = (t0, t1)

    @pl.when(cid == 0)
    def _():
        _spmm_one(a_hbm, src2_hbm, dst2_hbm, z_hbm, s1_hbm,
                  sblk, dblk, rows, acc, isem, gsems, ssems, sid)

    @pl.when(cid == 1)
    def _():
        _spmm_one(b_hbm, src2_hbm, dst2_hbm, z_hbm, s2_hbm,
                  sblk, dblk, rows, acc, isem, gsems, ssems, sid)


def _sc_spmm2(a, b, src2, dst2):
    mesh = plsc.VectorSubcoreMesh(core_axis_name="c", subcore_axis_name="s")
    zeros = jnp.zeros((RPT, D), jnp.float32)
    bufs = [pltpu.VMEM((SKC, D), jnp.float32) for _ in range(SPB)]
    sems = [pltpu.SemaphoreType.DMA for _ in range(1 + 2 * SPB)]
    kern = pl.kernel(
        _spmm_body,
        out_type=[jax.ShapeDtypeStruct((NPAD, D), jnp.float32),
                  jax.ShapeDtypeStruct((NPAD, D), jnp.float32)],
        mesh=mesh,
        scratch_types=[pltpu.VMEM((GRP, SKC), jnp.int32),
                       pltpu.VMEM((GRP, SKC), jnp.int32)]
        + bufs + [pltpu.VMEM_SHARED((NPAD, D), jnp.float32)] + sems,
    )
    return kern(a, b, src2, dst2, zeros)


RB = 1000


def _dinv_from(dega_blk, degb_blk):
    return lax.rsqrt(dega_blk + degb_blk + 1.0)


def _prep_body(feat, w1, emb, dega, degb, a_out, b_out):
    dinv = _dinv_from(dega[...], degb[...])
    xw = lax.dot_general(feat[...], w1[...], (((1,), (1,)), ((), ())),
                         preferred_element_type=jnp.float32)
    a_out[...] = dinv * xw
    b_out[...] = dinv * emb[...]


def _tc_prep(features, W1, embeddings, dega, degb):
    grid = (N // RB,)
    return pl.pallas_call(
        _prep_body,
        grid=grid,
        in_specs=[
            pl.BlockSpec((RB, D), lambda i: (i, 0)),
            pl.BlockSpec((D, D), lambda i: (0, 0)),
            pl.BlockSpec((RB, D), lambda i: (i, 0)),
            pl.BlockSpec((RB, D), lambda i: (i, 0)),
            pl.BlockSpec((RB, D), lambda i: (i, 0)),
        ],
        out_specs=[pl.BlockSpec((RB, D), lambda i: (i, 0)),
                   pl.BlockSpec((RB, D), lambda i: (i, 0))],
        out_shape=[jax.ShapeDtypeStruct((N, D), jnp.float32),
                   jax.ShapeDtypeStruct((N, D), jnp.float32)],
    )(features, W1, embeddings, dega, degb)


def _finish_body(s1, s2, a, b, dega, degb, f, b1, out):
    ftf = lax.dot_general(f[...], f[...], (((0,), (0,)), ((), ())),
                          preferred_element_type=jnp.float32)
    p = ftf / (jnp.sqrt(jnp.sum(ftf * ftf)) + 1e-5)
    dinv = _dinv_from(dega[...], degb[...])
    y = dinv * (s2[...] + b[...])
    h = jnp.maximum(dinv * (s1[...] + a[...]) + b1[...], 0.0)
    out[...] = 0.95 * lax.dot_general(y, p, (((1,), (0,)), ((), ())),
                                      preferred_element_type=jnp.float32) + h


def _tc_finish(s1, s2, a, b, dega, degb, F_mat, b1row):
    grid = (N // RB,)
    blk = pl.BlockSpec((RB, D), lambda i: (i, 0))
    return pl.pallas_call(
        _finish_body,
        grid=grid,
        in_specs=[blk, blk, blk, blk, blk, blk,
                  pl.BlockSpec((D, D), lambda i: (0, 0)),
                  pl.BlockSpec((1, D), lambda i: (0, 0))],
        out_specs=blk,
        out_shape=jax.ShapeDtypeStruct((N, D), jnp.float32),
    )(s1, s2, a, b, dega, degb, F_mat, b1row)


def kernel(features, edge_index, W1, b1, F_mat, embeddings):
    src = edge_index[0]
    dst = edge_index[1]
    pad = PADE - E
    srcp = jnp.concatenate([src, jnp.zeros((pad,), jnp.int32)])
    dstp = jnp.concatenate([dst, jnp.full((pad,), NPAD - 1, jnp.int32)])
    src2 = srcp.reshape(PADE // SKC, SKC)
    dst2 = dstp.reshape(PADE // SKC, SKC)
    dst3d = dstp.reshape(NC * NS, DG_CH, KC)
    degp = _sc_degree(dst3d)
    dega = degp[0, :N]
    degb = degp[1, :N]
    a, b = _tc_prep(features, W1, embeddings, dega, degb)
    s1, s2 = _sc_spmm2(a, b, src2, dst2)
    return _tc_finish(s1[:N], s2[:N], a, b, dega, degb, F_mat,
                      jnp.reshape(b1, (1, D)))


# speedup vs baseline: 1.1937x; 1.1937x over previous
import functools

import jax
import jax.numpy as jnp
from jax import lax
from jax.experimental import pallas as pl
from jax.experimental.pallas import tpu as pltpu
from jax.experimental.pallas import tpu_sc as plsc

N = 10000
E = 320000
D = 128

NC = 2
NS = 16
NPAD = 10240
RPT = NPAD // NS
K = 80

KC = 128
RBUF = 4
SPB = 2
PADE = 327680

DG_CH = PADE // (NC * NS * KC)
SP_CH = PADE // (NS * KC)


def _deg_body(dst3_hbm, ones_hbm, z_hbm, degp_hbm, dall, ones_v, acc,
              s0, s1, s2, s3):
    cid = lax.axis_index("c")
    sid = lax.axis_index("s")
    wid = cid * NS + sid
    sems = (s0, s1, s2, s3)
    pltpu.sync_copy(ones_hbm, ones_v)
    pltpu.sync_copy(dst3_hbm.at[wid], dall)
    pltpu.sync_copy(z_hbm, acc.at[pl.ds(sid * RPT, RPT)])
    plsc.subcore_barrier()

    def outer(j, _):
        descs = []
        for b in range(RBUF):
            i = j * RBUF + b
            descs.append(pltpu.async_copy(ones_v, acc.at[dall.at[i]],
                                          sems[b], add=True))
        for d in descs:
            d.wait()
        return 0

    lax.fori_loop(0, DG_CH // RBUF, outer, 0)
    plsc.subcore_barrier()
    pltpu.sync_copy(acc.at[pl.ds(sid * RPT, RPT)],
                    degp_hbm.at[cid, pl.ds(sid * RPT, RPT)])


def _sc_degree(dst3):
    mesh = plsc.VectorSubcoreMesh(core_axis_name="c", subcore_axis_name="s")
    ones = jnp.ones((KC, D), jnp.float32)
    zeros = jnp.zeros((RPT, D), jnp.float32)
    kern = pl.kernel(
        _deg_body,
        out_type=jax.ShapeDtypeStruct((NC, NPAD, D), jnp.float32),
        mesh=mesh,
        scratch_types=[
            pltpu.VMEM((DG_CH, KC), jnp.int32),
            pltpu.VMEM((KC, D), jnp.float32),
            pltpu.VMEM_SHARED((NPAD, D), jnp.float32),
            pltpu.SemaphoreType.DMA,
            pltpu.SemaphoreType.DMA,
            pltpu.SemaphoreType.DMA,
            pltpu.SemaphoreType.DMA,
        ],
    )
    return kern(dst3, ones, zeros)


K80 = 80
SP_EPT = E // NS
SP_IT = SP_EPT // K80


def _spmm_one(tab_hbm, src_hbm, dst_hbm, z_hbm, out_hbm,
              sidx, didx0, didx1, rows, acc, gsem, ssems, sid):
    pltpu.sync_copy(z_hbm, acc.at[pl.ds(sid * RPT, RPT)])
    plsc.subcore_barrier()
    didx = (didx0, didx1)

    def step(j, _):
        sds = [None, None]
        for r in range(2):
            e0 = sid * SP_EPT + (2 * j + r) * K80
            pltpu.sync_copy(src_hbm.at[pl.ds(e0, K80)], sidx)
            pltpu.sync_copy(dst_hbm.at[pl.ds(e0, K80)], didx[r])
            pltpu.async_copy(tab_hbm.at[sidx], rows[r], gsem).wait()
            sds[r] = pltpu.async_copy(rows[r], acc.at[didx[r]],
                                      ssems[r], add=True)
        for d in sds:
            d.wait()
        return 0

    lax.fori_loop(0, SP_IT // 2, step, 0)
    plsc.subcore_barrier()
    pltpu.sync_copy(acc.at[pl.ds(sid * RPT, RPT)],
                    out_hbm.at[pl.ds(sid * RPT, RPT)])


def _spmm_body(a_hbm, b_hbm, src_hbm, dst_hbm, z_hbm, s1_hbm, s2_hbm,
               sidx, didx0, didx1, r0, r1, acc, gsem, t0, t1):
    cid = lax.axis_index("c")
    sid = lax.axis_index("s")
    rows = (r0, r1)
    ssems = (t0, t1)

    @pl.when(cid == 0)
    def _():
        _spmm_one(a_hbm, src_hbm, dst_hbm, z_hbm, s1_hbm,
                  sidx, didx0, didx1, rows, acc, gsem, ssems, sid)

    @pl.when(cid == 1)
    def _():
        _spmm_one(b_hbm, src_hbm, dst_hbm, z_hbm, s2_hbm,
                  sidx, didx0, didx1, rows, acc, gsem, ssems, sid)


def _sc_spmm2(a, b, src, dst):
    mesh = plsc.VectorSubcoreMesh(core_axis_name="c", subcore_axis_name="s")
    zeros = jnp.zeros((RPT, D), jnp.float32)
    kern = pl.kernel(
        _spmm_body,
        out_type=[jax.ShapeDtypeStruct((NPAD, D), jnp.float32),
                  jax.ShapeDtypeStruct((NPAD, D), jnp.float32)],
        mesh=mesh,
        scratch_types=[
            pltpu.VMEM((K80,), jnp.int32),
            pltpu.VMEM((K80,), jnp.int32),
            pltpu.VMEM((K80,), jnp.int32),
            pltpu.VMEM((K80, D), jnp.float32),
            pltpu.VMEM((K80, D), jnp.float32),
            pltpu.VMEM_SHARED((NPAD, D), jnp.float32),
            pltpu.SemaphoreType.DMA,
            pltpu.SemaphoreType.DMA,
            pltpu.SemaphoreType.DMA,
        ],
    )
    return kern(a, b, src, dst, zeros)


RB = 1000


def _dinv_from(dega_blk, degb_blk):
    return lax.rsqrt(dega_blk + degb_blk + 1.0)


def _prep_body(feat, w1, emb, dega, degb, a_out, b_out):
    dinv = _dinv_from(dega[...], degb[...])
    xw = lax.dot_general(feat[...], w1[...], (((1,), (1,)), ((), ())),
                         preferred_element_type=jnp.float32)
    a_out[...] = dinv * xw
    b_out[...] = dinv * emb[...]


def _tc_prep(features, W1, embeddings, dega, degb):
    grid = (N // RB,)
    return pl.pallas_call(
        _prep_body,
        grid=grid,
        in_specs=[
            pl.BlockSpec((RB, D), lambda i: (i, 0)),
            pl.BlockSpec((D, D), lambda i: (0, 0)),
            pl.BlockSpec((RB, D), lambda i: (i, 0)),
            pl.BlockSpec((RB, D), lambda i: (i, 0)),
            pl.BlockSpec((RB, D), lambda i: (i, 0)),
        ],
        out_specs=[pl.BlockSpec((RB, D), lambda i: (i, 0)),
                   pl.BlockSpec((RB, D), lambda i: (i, 0))],
        out_shape=[jax.ShapeDtypeStruct((N, D), jnp.float32),
                   jax.ShapeDtypeStruct((N, D), jnp.float32)],
    )(features, W1, embeddings, dega, degb)


def _finish_body(s1, s2, a, b, dega, degb, f, b1, out):
    ftf = lax.dot_general(f[...], f[...], (((0,), (0,)), ((), ())),
                          preferred_element_type=jnp.float32)
    p = ftf / (jnp.sqrt(jnp.sum(ftf * ftf)) + 1e-5)
    dinv = _dinv_from(dega[...], degb[...])
    y = dinv * (s2[...] + b[...])
    h = jnp.maximum(dinv * (s1[...] + a[...]) + b1[...], 0.0)
    out[...] = 0.95 * lax.dot_general(y, p, (((1,), (0,)), ((), ())),
                                      preferred_element_type=jnp.float32) + h


def _tc_finish(s1, s2, a, b, dega, degb, F_mat, b1row):
    grid = (N // RB,)
    blk = pl.BlockSpec((RB, D), lambda i: (i, 0))
    return pl.pallas_call(
        _finish_body,
        grid=grid,
        in_specs=[blk, blk, blk, blk, blk, blk,
                  pl.BlockSpec((D, D), lambda i: (0, 0)),
                  pl.BlockSpec((1, D), lambda i: (0, 0))],
        out_specs=blk,
        out_shape=jax.ShapeDtypeStruct((N, D), jnp.float32),
    )(s1, s2, a, b, dega, degb, F_mat, b1row)


def kernel(features, edge_index, W1, b1, F_mat, embeddings):
    src = edge_index[0]
    dst = edge_index[1]
    pad = PADE - E
    srcp = jnp.concatenate([src, jnp.zeros((pad,), jnp.int32)])
    dstp = jnp.concatenate([dst, jnp.full((pad,), NPAD - 1, jnp.int32)])
    dst3d = dstp.reshape(NC * NS, DG_CH, KC)
    degp = _sc_degree(dst3d)
    dega = degp[0, :N]
    degb = degp[1, :N]
    a, b = _tc_prep(features, W1, embeddings, dega, degb)
    s1, s2 = _sc_spmm2(a, b, src, dst)
    return _tc_finish(s1[:N], s2[:N], a, b, dega, degb, F_mat,
                      jnp.reshape(b1, (1, D)))


# speedup vs baseline: 1.4536x; 1.2178x over previous
import functools

import jax
import jax.numpy as jnp
from jax import lax
from jax.experimental import pallas as pl
from jax.experimental.pallas import tpu as pltpu
from jax.experimental.pallas import tpu_sc as plsc

N = 10000
E = 320000
D = 128

NC = 2
NS = 16
NPAD = 10240
RPT = NPAD // NS
K = 80

KC = 128
RBUF = 4
SPB = 2
PADE = 327680

DG_CH = PADE // (NC * NS * KC)
SP_CH = PADE // (NS * KC)


def _deg_body(dst3_hbm, ones_hbm, z_hbm, degp_hbm, dall, ones_v, acc,
              s0, s1, s2, s3):
    cid = lax.axis_index("c")
    sid = lax.axis_index("s")
    wid = cid * NS + sid
    sems = (s0, s1, s2, s3)
    pltpu.sync_copy(ones_hbm, ones_v)
    pltpu.sync_copy(dst3_hbm.at[wid], dall)
    pltpu.sync_copy(z_hbm, acc.at[pl.ds(sid * RPT, RPT)])
    plsc.subcore_barrier()

    def outer(j, _):
        descs = []
        for b in range(RBUF):
            i = j * RBUF + b
            descs.append(pltpu.async_copy(ones_v, acc.at[dall.at[i]],
                                          sems[b], add=True))
        for d in descs:
            d.wait()
        return 0

    lax.fori_loop(0, DG_CH // RBUF, outer, 0)
    plsc.subcore_barrier()
    pltpu.sync_copy(acc.at[pl.ds(sid * RPT, RPT)],
                    degp_hbm.at[cid, pl.ds(sid * RPT, RPT)])


def _sc_degree(dst3):
    mesh = plsc.VectorSubcoreMesh(core_axis_name="c", subcore_axis_name="s")
    ones = jnp.ones((KC, D), jnp.float32)
    zeros = jnp.zeros((RPT, D), jnp.float32)
    kern = pl.kernel(
        _deg_body,
        out_type=jax.ShapeDtypeStruct((NC, NPAD, D), jnp.float32),
        mesh=mesh,
        scratch_types=[
            pltpu.VMEM((DG_CH, KC), jnp.int32),
            pltpu.VMEM((KC, D), jnp.float32),
            pltpu.VMEM_SHARED((NPAD, D), jnp.float32),
            pltpu.SemaphoreType.DMA,
            pltpu.SemaphoreType.DMA,
            pltpu.SemaphoreType.DMA,
            pltpu.SemaphoreType.DMA,
        ],
    )
    return kern(dst3, ones, zeros)


K80 = 80
SP_EPT = E // NS
SP_IT = SP_EPT // K80


def _spmm_one(tab_hbm, src_hbm, dst_hbm, z_hbm, out_hbm,
              sidx0, sidx1, didx0, didx1, rows, acc, gsems, ssems, sid):
    pltpu.sync_copy(z_hbm, acc.at[pl.ds(sid * RPT, RPT)])
    plsc.subcore_barrier()
    sidx = (sidx0, sidx1)
    didx = (didx0, didx1)

    def step(j, _):
        gds = [None, None]
        sds = [None, None]
        for r in range(2):
            e0 = sid * SP_EPT + (2 * j + r) * K80
            pltpu.sync_copy(src_hbm.at[pl.ds(e0, K80)], sidx[r])
            pltpu.sync_copy(dst_hbm.at[pl.ds(e0, K80)], didx[r])
            gds[r] = pltpu.async_copy(tab_hbm.at[sidx[r]], rows[r], gsems[r])
        for r in range(2):
            gds[r].wait()
            sds[r] = pltpu.async_copy(rows[r], acc.at[didx[r]],
                                      ssems[r], add=True)
        for d in sds:
            d.wait()
        return 0

    lax.fori_loop(0, SP_IT // 2, step, 0)
    plsc.subcore_barrier()
    pltpu.sync_copy(acc.at[pl.ds(sid * RPT, RPT)],
                    out_hbm.at[pl.ds(sid * RPT, RPT)])


def _spmm_body(a_hbm, b_hbm, src_hbm, dst_hbm, z_hbm, s1_hbm, s2_hbm,
               sidx0, sidx1, didx0, didx1, r0, r1, acc, g0, g1, t0, t1):
    cid = lax.axis_index("c")
    sid = lax.axis_index("s")
    rows = (r0, r1)
    gsems = (g0, g1)
    ssems = (t0, t1)

    @pl.when(cid == 0)
    def _():
        _spmm_one(a_hbm, src_hbm, dst_hbm, z_hbm, s1_hbm,
                  sidx0, sidx1, didx0, didx1, rows, acc, gsems, ssems, sid)

    @pl.when(cid == 1)
    def _():
        _spmm_one(b_hbm, src_hbm, dst_hbm, z_hbm, s2_hbm,
                  sidx0, sidx1, didx0, didx1, rows, acc, gsems, ssems, sid)


def _sc_spmm2(a, b, src, dst):
    mesh = plsc.VectorSubcoreMesh(core_axis_name="c", subcore_axis_name="s")
    zeros = jnp.zeros((RPT, D), jnp.float32)
    kern = pl.kernel(
        _spmm_body,
        out_type=[jax.ShapeDtypeStruct((NPAD, D), jnp.float32),
                  jax.ShapeDtypeStruct((NPAD, D), jnp.float32)],
        mesh=mesh,
        scratch_types=[
            pltpu.VMEM((K80,), jnp.int32),
            pltpu.VMEM((K80,), jnp.int32),
            pltpu.VMEM((K80,), jnp.int32),
            pltpu.VMEM((K80,), jnp.int32),
            pltpu.VMEM((K80, D), jnp.float32),
            pltpu.VMEM((K80, D), jnp.float32),
            pltpu.VMEM_SHARED((NPAD, D), jnp.float32),
            pltpu.SemaphoreType.DMA,
            pltpu.SemaphoreType.DMA,
            pltpu.SemaphoreType.DMA,
            pltpu.SemaphoreType.DMA,
        ],
    )
    return kern(a, b, src, dst, zeros)


RB = 1000


def _dinv_from(dega_blk, degb_blk):
    return lax.rsqrt(dega_blk + degb_blk + 1.0)


def _prep_body(feat, w1, emb, dega, degb, a_out, b_out):
    dinv = _dinv_from(dega[...], degb[...])
    xw = lax.dot_general(feat[...], w1[...], (((1,), (1,)), ((), ())),
                         preferred_element_type=jnp.float32)
    a_out[...] = dinv * xw
    b_out[...] = dinv * emb[...]


def _tc_prep(features, W1, embeddings, dega, degb):
    grid = (N // RB,)
    return pl.pallas_call(
        _prep_body,
        grid=grid,
        in_specs=[
            pl.BlockSpec((RB, D), lambda i: (i, 0)),
            pl.BlockSpec((D, D), lambda i: (0, 0)),
            pl.BlockSpec((RB, D), lambda i: (i, 0)),
            pl.BlockSpec((RB, D), lambda i: (i, 0)),
            pl.BlockSpec((RB, D), lambda i: (i, 0)),
        ],
        out_specs=[pl.BlockSpec((RB, D), lambda i: (i, 0)),
                   pl.BlockSpec((RB, D), lambda i: (i, 0))],
        out_shape=[jax.ShapeDtypeStruct((N, D), jnp.float32),
                   jax.ShapeDtypeStruct((N, D), jnp.float32)],
    )(features, W1, embeddings, dega, degb)


def _finish_body(s1, s2, a, b, dega, degb, f, b1, out):
    ftf = lax.dot_general(f[...], f[...], (((0,), (0,)), ((), ())),
                          preferred_element_type=jnp.float32)
    p = ftf / (jnp.sqrt(jnp.sum(ftf * ftf)) + 1e-5)
    dinv = _dinv_from(dega[...], degb[...])
    y = dinv * (s2[...] + b[...])
    h = jnp.maximum(dinv * (s1[...] + a[...]) + b1[...], 0.0)
    out[...] = 0.95 * lax.dot_general(y, p, (((1,), (0,)), ((), ())),
                                      preferred_element_type=jnp.float32) + h


def _tc_finish(s1, s2, a, b, dega, degb, F_mat, b1row):
    grid = (N // RB,)
    blk = pl.BlockSpec((RB, D), lambda i: (i, 0))
    return pl.pallas_call(
        _finish_body,
        grid=grid,
        in_specs=[blk, blk, blk, blk, blk, blk,
                  pl.BlockSpec((D, D), lambda i: (0, 0)),
                  pl.BlockSpec((1, D), lambda i: (0, 0))],
        out_specs=blk,
        out_shape=jax.ShapeDtypeStruct((N, D), jnp.float32),
    )(s1, s2, a, b, dega, degb, F_mat, b1row)


def kernel(features, edge_index, W1, b1, F_mat, embeddings):
    src = edge_index[0]
    dst = edge_index[1]
    pad = PADE - E
    srcp = jnp.concatenate([src, jnp.zeros((pad,), jnp.int32)])
    dstp = jnp.concatenate([dst, jnp.full((pad,), NPAD - 1, jnp.int32)])
    dst3d = dstp.reshape(NC * NS, DG_CH, KC)
    degp = _sc_degree(dst3d)
    dega = degp[0, :N]
    degb = degp[1, :N]
    a, b = _tc_prep(features, W1, embeddings, dega, degb)
    s1, s2 = _sc_spmm2(a, b, src, dst)
    return _tc_finish(s1[:N], s2[:N], a, b, dega, degb, F_mat,
                      jnp.reshape(b1, (1, D)))
